# baseline (speedup 1.0000x reference)
import functools

import jax
import jax.numpy as jnp
from jax import lax
from jax.experimental import pallas as pl
from jax.experimental.pallas import tpu as pltpu
from jax.experimental.pallas import tpu_sc as plsc

NUM_ROWS = 21
D_MODEL = 128

_NC = 2
_NS = 16
_NW = _NC * _NS

_B = 4096 * 100
_B_PER_W = _B // _NW
_CHUNK = 128
_N_CHUNKS = _B_PER_W // _CHUNK
_N_PAIRS = _N_CHUNKS // 2


def _emb_body(idx_hbm, table_hbm, out_hbm, idx_v, table_sp,
              grp_a, grp_b, ga, gb, wa, wb):
    sid = lax.axis_index("s")
    wid = sid * _NC + lax.axis_index("c")

    pl.when(sid == 0)(lambda: pltpu.sync_copy(table_hbm, table_sp))

    pltpu.sync_copy(idx_hbm.at[wid], idx_v)
    plsc.subcore_barrier()

    def fire_gathers(p, grp, sem):
        pltpu.make_async_copy(table_sp.at[idx_v.at[2 * p]], grp.at[0], sem).start()
        pltpu.make_async_copy(table_sp.at[idx_v.at[2 * p + 1]], grp.at[1], sem).start()

    def wait_gathers(grp, sem):
        pltpu.make_async_copy(grp, out_hbm.at[wid, 0], sem).wait()

    def fire_write(p, grp, sem):
        pltpu.make_async_copy(grp, out_hbm.at[wid, p], sem).start()

    def wait_write(grp, sem):
        pltpu.make_async_copy(grp, out_hbm.at[wid, 0], sem).wait()

    def step(p, cur, fire_next, first=False):
        grp, g_sem, w_sem = (grp_a, ga, wa) if cur == 0 else (grp_b, gb, wb)
        o_grp, o_g, o_w = (grp_b, gb, wb) if cur == 0 else (grp_a, ga, wa)
        wait_gathers(grp, g_sem)
        fire_write(p, grp, w_sem)
        if not first:
            wait_write(o_grp, o_w)
        if fire_next:
            fire_gathers(p + 1, o_grp, o_g)

    fire_gathers(0, grp_a, ga)
    step(0, 0, True, first=True)

    def body(i, carry):
        step(2 * i + 1, 1, True)
        step(2 * i + 2, 0, True)
        return carry

    lax.fori_loop(0, (_N_PAIRS - 2) // 2, body, 0)

    step(_N_PAIRS - 1, 1, False)
    wait_write(grp_b, wb)


_emb = functools.partial(
    pl.kernel,
    out_type=jax.ShapeDtypeStruct((_NW, _N_PAIRS, 2, _CHUNK, D_MODEL), jnp.float32),
    mesh=plsc.VectorSubcoreMesh(core_axis_name="c", subcore_axis_name="s"),
    scratch_types=[
        pltpu.VMEM((_N_CHUNKS, _CHUNK), jnp.int32),
        pltpu.VMEM_SHARED((NUM_ROWS, D_MODEL), jnp.float32),
        pltpu.VMEM((2, _CHUNK, D_MODEL), jnp.float32),
        pltpu.VMEM((2, _CHUNK, D_MODEL), jnp.float32),
        pltpu.SemaphoreType.DMA,
        pltpu.SemaphoreType.DMA,
        pltpu.SemaphoreType.DMA,
        pltpu.SemaphoreType.DMA,
    ],
)(_emb_body)


def kernel(sensor_indices, embedding_table):
    b, t = sensor_indices.shape
    idx = sensor_indices.T.reshape(_NW, _N_CHUNKS, _CHUNK).astype(jnp.int32)
    out = _emb(idx, embedding_table)
    return out.reshape(t, b, D_MODEL).transpose(1, 0, 2)


# speedup vs baseline: 1.0382x; 1.0382x over previous
import functools

import jax
import jax.numpy as jnp
from jax import lax
from jax.experimental import pallas as pl
from jax.experimental.pallas import tpu as pltpu
from jax.experimental.pallas import tpu_sc as plsc

NUM_ROWS = 21
D_MODEL = 128

_NC = 2
_NS = 16
_NW = _NC * _NS

_B = 4096 * 100
_B_PER_W = _B // _NW
_CHUNK = 128
_N_CHUNKS = _B_PER_W // _CHUNK
_NBUF = 5
_FIRE_AHEAD = 3
_N_OUTER = _N_CHUNKS // _NBUF


def _emb_body(idx_hbm, table_hbm, out_hbm, idx_v, table_sp,
              b0, b1, b2, b3, b4,
              gs0, gs1, gs2, gs3, gs4, ws0, ws1, ws2, ws3, ws4):
    sid = lax.axis_index("s")
    wid = sid * _NC + lax.axis_index("c")
    bufs = (b0, b1, b2, b3, b4)
    gs = (gs0, gs1, gs2, gs3, gs4)
    ws = (ws0, ws1, ws2, ws3, ws4)

    pl.when(sid == 0)(lambda: pltpu.sync_copy(table_hbm, table_sp))

    pltpu.sync_copy(idx_hbm.at[wid], idx_v)
    plsc.subcore_barrier()

    def fire_gather(s, buf, sem):
        pltpu.make_async_copy(table_sp.at[idx_v.at[s]], buf, sem).start()

    def wait_gather(buf, sem):
        pltpu.make_async_copy(table_sp.at[idx_v.at[0]], buf, sem).wait()

    def fire_write(s, buf, sem):
        pltpu.make_async_copy(buf, out_hbm.at[wid, s], sem).start()

    def wait_write(buf, sem):
        pltpu.make_async_copy(buf, out_hbm.at[wid, 0], sem).wait()

    def body(i, carry):
        for b in range(_NBUF):
            j = _NBUF * i + b
            kb = (b + _FIRE_AHEAD) % _NBUF
            k = j + _FIRE_AHEAD
            wait_gather(bufs[b], gs[b])
            fire_write(j, bufs[b], ws[b])
            if b < _NBUF - _FIRE_AHEAD:
                pl.when(i >= 1)(lambda: wait_write(bufs[kb], ws[kb]))
                fire_gather(k, bufs[kb], gs[kb])
            else:
                def wait_then_fire(kb=kb, k=k):
                    wait_write(bufs[kb], ws[kb])
                    fire_gather(k, bufs[kb], gs[kb])
                pl.when(i < _N_OUTER - 1)(wait_then_fire)
        return carry

    for s in range(_FIRE_AHEAD):
        fire_gather(s, bufs[s], gs[s])
    lax.fori_loop(0, _N_OUTER, body, 0)
    for b in range(_NBUF):
        wait_write(bufs[b], ws[b])


_emb = functools.partial(
    pl.kernel,
    out_type=jax.ShapeDtypeStruct((_NW, _N_CHUNKS, _CHUNK, D_MODEL), jnp.float32),
    mesh=plsc.VectorSubcoreMesh(core_axis_name="c", subcore_axis_name="s"),
    scratch_types=[
        pltpu.VMEM((_N_CHUNKS, _CHUNK), jnp.int32),
        pltpu.VMEM_SHARED((NUM_ROWS, D_MODEL), jnp.float32),
        pltpu.VMEM((_CHUNK, D_MODEL), jnp.float32),
        pltpu.VMEM((_CHUNK, D_MODEL), jnp.float32),
        pltpu.VMEM((_CHUNK, D_MODEL), jnp.float32),
        pltpu.VMEM((_CHUNK, D_MODEL), jnp.float32),
        pltpu.VMEM((_CHUNK, D_MODEL), jnp.float32),
        pltpu.SemaphoreType.DMA,
        pltpu.SemaphoreType.DMA,
        pltpu.SemaphoreType.DMA,
        pltpu.SemaphoreType.DMA,
        pltpu.SemaphoreType.DMA,
        pltpu.SemaphoreType.DMA,
        pltpu.SemaphoreType.DMA,
        pltpu.SemaphoreType.DMA,
        pltpu.SemaphoreType.DMA,
        pltpu.SemaphoreType.DMA,
    ],
)(_emb_body)


def kernel(sensor_indices, embedding_table):
    b, t = sensor_indices.shape
    idx = sensor_indices.T.reshape(_NW, _N_CHUNKS, _CHUNK).astype(jnp.int32)
    out = _emb(idx, embedding_table)
    return out.reshape(t, b, D_MODEL).transpose(1, 0, 2)
